# baseline (speedup 1.0000x reference)
import functools

import jax
import jax.numpy as jnp
from jax import lax
from jax.experimental import pallas as pl
from jax.experimental.pallas import tpu as pltpu
from jax.experimental.pallas import tpu_sc as plsc

INPUT_DIM = 2048
K = 512
CURRENT_TEMP = 5.0
ROWS_PER_BLOCK = 512
TC1_BLOCKS = 6
ONE_BITS = 0x3F800000

_NTILES = 16
_PER_TILE = INPUT_DIM // _NTILES
_NVREG = INPUT_DIM // 16


def _sc_mask_body(soft_hbm, mask_hbm, soft_v, out_v):
    wid = lax.axis_index("s")
    pltpu.sync_copy(soft_hbm, soft_v)

    def count_ge(t):
        def cbody(j, acc):
            for u in range(8):
                v = soft_v[pl.ds((j * 8 + u) * 16, 16)]
                b = lax.bitcast_convert_type(v, jnp.int32)
                acc = acc + jnp.where(b >= t, jnp.int32(1), jnp.int32(0))
            return acc

        acc = lax.fori_loop(0, _NVREG // 8, cbody,
                            jnp.zeros((16,), jnp.int32))
        return jnp.sum(acc)

    def bstep(_, carry):
        lo, hi = carry
        mid = lo + (hi - lo + 1) // 2
        pred = count_ge(mid) >= K
        return (jnp.where(pred, mid, lo), jnp.where(pred, hi, mid - 1))

    lo, _ = lax.fori_loop(0, 31, bstep,
                          (jnp.int32(0), jnp.int32(ONE_BITS)))
    base = wid * _PER_TILE
    for j in range(_PER_TILE // 16):
        v = soft_v[pl.ds(base + j * 16, 16)]
        b = lax.bitcast_convert_type(v, jnp.int32)
        out_v[pl.ds(j * 16, 16)] = jnp.where(
            b >= lo, jnp.float32(1.0), jnp.float32(0.0))
    pltpu.sync_copy(out_v, mask_hbm.at[pl.ds(base, _PER_TILE)])


_sc_mask = functools.partial(
    pl.kernel,
    out_type=jax.ShapeDtypeStruct((INPUT_DIM,), jnp.float32),
    mesh=plsc.VectorSubcoreMesh(core_axis_name="c", subcore_axis_name="s",
                                num_cores=1),
    compiler_params=pltpu.CompilerParams(needs_layout_passes=False,
                                         skip_device_barrier=True),
    scratch_types=[
        pltpu.VMEM((INPUT_DIM,), jnp.float32),
        pltpu.VMEM((_PER_TILE,), jnp.float32),
    ],
)(_sc_mask_body)


def _tc1_body(soft_ref, x_ref, out_ref, mask_ref):
    @pl.when(pl.program_id(0) == 0)
    def _compute_mask():
        soft = soft_ref[...]
        bits = lax.bitcast_convert_type(soft, jnp.int32)

        def step(_, carry):
            lo, hi = carry
            mid = lo + (hi - lo + 1) // 2
            cnt = jnp.sum((bits >= mid).astype(jnp.int32))
            pred = cnt >= K
            return jnp.where(pred, mid, lo), jnp.where(pred, hi, mid - 1)

        lo, _ = lax.fori_loop(
            0, 31, step, (jnp.int32(0), jnp.int32(ONE_BITS)))
        mask_ref[...] = (bits >= lo).astype(jnp.float32)

    out_ref[...] = x_ref[...] * mask_ref[...]


def _tc2_body(mask_ref, x_ref, prev_ref, out_ref):
    del prev_ref
    out_ref[...] = x_ref[...] * mask_ref[...]


def kernel(x, logits):
    noise_key = jax.random.key(42)
    u = jax.random.uniform(noise_key, logits.shape, dtype=logits.dtype)
    noise = -jnp.log(-jnp.log(u + 1e-20) + 1e-20)
    soft_probs = jax.nn.sigmoid((logits + noise) / CURRENT_TEMP)
    soft2d = soft_probs.reshape(1, INPUT_DIM)

    b, s, d = x.shape
    rows = b * s
    x2 = x.reshape(rows, d)
    nblocks = rows // ROWS_PER_BLOCK

    out1 = pl.pallas_call(
        _tc1_body,
        grid=(TC1_BLOCKS,),
        in_specs=[
            pl.BlockSpec((1, d), lambda i: (0, 0)),
            pl.BlockSpec((ROWS_PER_BLOCK, d), lambda i: (i, 0)),
        ],
        out_specs=pl.BlockSpec((ROWS_PER_BLOCK, d), lambda i: (i, 0)),
        out_shape=jax.ShapeDtypeStruct((rows, d), x.dtype),
        scratch_shapes=[pltpu.VMEM((1, d), jnp.float32)],
    )(soft2d, x2)

    mask = _sc_mask(soft_probs).reshape(1, INPUT_DIM)

    tc2_block = 2 * ROWS_PER_BLOCK
    tc2_off = TC1_BLOCKS * ROWS_PER_BLOCK // tc2_block
    out = pl.pallas_call(
        _tc2_body,
        grid=((rows - TC1_BLOCKS * ROWS_PER_BLOCK) // tc2_block,),
        in_specs=[
            pl.BlockSpec((1, d), lambda i: (0, 0)),
            pl.BlockSpec((tc2_block, d),
                         lambda i: (i + tc2_off, 0)),
            pl.BlockSpec(memory_space=pl.ANY),
        ],
        out_specs=pl.BlockSpec((tc2_block, d),
                               lambda i: (i + tc2_off, 0)),
        out_shape=jax.ShapeDtypeStruct((rows, d), x.dtype),
        input_output_aliases={2: 0},
    )(mask, x2, out1)
    return out.reshape(b, s, d)


# speedup vs baseline: 1.0000x; 1.0000x over previous
import functools

import jax
import jax.numpy as jnp
from jax import lax
from jax.experimental import pallas as pl
from jax.experimental.pallas import tpu as pltpu
from jax.experimental.pallas import tpu_sc as plsc

INPUT_DIM = 2048
K = 512
CURRENT_TEMP = 5.0
ROWS_PER_BLOCK = 512
TC1_BLOCKS = 6
ONE_BITS = 0x3F800000

_NTILES = 16
_PER_TILE = INPUT_DIM // _NTILES
_NVREG = INPUT_DIM // 16


def _sc_mask_body(soft_hbm, mask_hbm, soft_v, out_v):
    wid = lax.axis_index("s")
    pltpu.sync_copy(soft_hbm, soft_v)

    def count_ge(t):
        def cbody(j, acc):
            for u in range(8):
                v = soft_v[pl.ds((j * 8 + u) * 16, 16)]
                b = lax.bitcast_convert_type(v, jnp.int32)
                acc = acc + jnp.where(b >= t, jnp.int32(1), jnp.int32(0))
            return acc

        acc = lax.fori_loop(0, _NVREG // 8, cbody,
                            jnp.zeros((16,), jnp.int32))
        return jnp.sum(acc)

    def bstep(_, carry):
        lo, hi = carry
        mid = lo + (hi - lo + 1) // 2
        pred = count_ge(mid) >= K
        return (jnp.where(pred, mid, lo), jnp.where(pred, hi, mid - 1))

    lo, _ = lax.fori_loop(0, 31, bstep,
                          (jnp.int32(0), jnp.int32(ONE_BITS)))
    base = wid * _PER_TILE
    for j in range(_PER_TILE // 16):
        v = soft_v[pl.ds(base + j * 16, 16)]
        b = lax.bitcast_convert_type(v, jnp.int32)
        out_v[pl.ds(j * 16, 16)] = jnp.where(
            b >= lo, jnp.float32(1.0), jnp.float32(0.0))
    pltpu.sync_copy(out_v, mask_hbm.at[pl.ds(base, _PER_TILE)])


_sc_mask = functools.partial(
    pl.kernel,
    out_type=jax.ShapeDtypeStruct((INPUT_DIM,), jnp.float32),
    mesh=plsc.VectorSubcoreMesh(core_axis_name="c", subcore_axis_name="s",
                                num_cores=1),
    compiler_params=pltpu.CompilerParams(needs_layout_passes=False),
    scratch_types=[
        pltpu.VMEM((INPUT_DIM,), jnp.float32),
        pltpu.VMEM((_PER_TILE,), jnp.float32),
    ],
)(_sc_mask_body)


def _tc1_body(soft_ref, x_ref, out_ref, mask_ref):
    @pl.when(pl.program_id(0) == 0)
    def _compute_mask():
        soft = soft_ref[...]
        bits = lax.bitcast_convert_type(soft, jnp.int32)

        def step(_, carry):
            lo, hi = carry
            mid = lo + (hi - lo + 1) // 2
            cnt = jnp.sum((bits >= mid).astype(jnp.int32))
            pred = cnt >= K
            return jnp.where(pred, mid, lo), jnp.where(pred, hi, mid - 1)

        lo, _ = lax.fori_loop(
            0, 31, step, (jnp.int32(0), jnp.int32(ONE_BITS)))
        mask_ref[...] = (bits >= lo).astype(jnp.float32)

    out_ref[...] = x_ref[...] * mask_ref[...]


def _tc2_body(mask_ref, x_ref, prev_ref, out_ref):
    del prev_ref
    out_ref[...] = x_ref[...] * mask_ref[...]


def kernel(x, logits):
    noise_key = jax.random.key(42)
    u = jax.random.uniform(noise_key, logits.shape, dtype=logits.dtype)
    noise = -jnp.log(-jnp.log(u + 1e-20) + 1e-20)
    soft_probs = jax.nn.sigmoid((logits + noise) / CURRENT_TEMP)
    soft2d = soft_probs.reshape(1, INPUT_DIM)

    b, s, d = x.shape
    rows = b * s
    x2 = x.reshape(rows, d)
    nblocks = rows // ROWS_PER_BLOCK

    out1 = pl.pallas_call(
        _tc1_body,
        grid=(TC1_BLOCKS,),
        in_specs=[
            pl.BlockSpec((1, d), lambda i: (0, 0)),
            pl.BlockSpec((ROWS_PER_BLOCK, d), lambda i: (i, 0)),
        ],
        out_specs=pl.BlockSpec((ROWS_PER_BLOCK, d), lambda i: (i, 0)),
        out_shape=jax.ShapeDtypeStruct((rows, d), x.dtype),
        scratch_shapes=[pltpu.VMEM((1, d), jnp.float32)],
    )(soft2d, x2)

    mask = _sc_mask(soft_probs).reshape(1, INPUT_DIM)

    tc2_block = 2 * ROWS_PER_BLOCK
    tc2_off = TC1_BLOCKS * ROWS_PER_BLOCK // tc2_block
    out = pl.pallas_call(
        _tc2_body,
        grid=((rows - TC1_BLOCKS * ROWS_PER_BLOCK) // tc2_block,),
        in_specs=[
            pl.BlockSpec((1, d), lambda i: (0, 0)),
            pl.BlockSpec((tc2_block, d),
                         lambda i: (i + tc2_off, 0)),
            pl.BlockSpec(memory_space=pl.ANY),
        ],
        out_specs=pl.BlockSpec((tc2_block, d),
                               lambda i: (i + tc2_off, 0)),
        out_shape=jax.ShapeDtypeStruct((rows, d), x.dtype),
        input_output_aliases={2: 0},
    )(mask, x2, out1)
    return out.reshape(b, s, d)
